# baseline (speedup 1.0000x reference)
import jax
import jax.numpy as jnp
from jax.experimental import pallas as pl
from jax.experimental.pallas import tpu as pltpu


def _body(x_ref, w_ref, b_ref, logits_ref, probs_ref):
    x = x_ref[...]
    w = w_ref[...]
    logits = jax.lax.dot_general(
        w, x,
        dimension_numbers=(((1,), (1,)), ((), ())),
        preferred_element_type=jnp.float32,
    ) + b_ref[...]
    logits_ref[...] = logits
    m = jnp.max(logits, axis=0, keepdims=True)
    e = jnp.exp(logits - m)
    probs_ref[...] = e / jnp.sum(e, axis=0, keepdims=True)


def _outer(x_hbm, w_hbm, b_hbm, logits_hbm, probs_hbm, *, tile, n, d, ne):
    pipeline = pltpu.emit_pipeline(
        _body,
        grid=(n // tile,),
        in_specs=[
            pl.BlockSpec((tile, d), lambda i: (i, 0),
                         pipeline_mode=pl.Buffered(buffer_count=4)),
            pl.BlockSpec((ne, d), lambda i: (0, 0)),
            pl.BlockSpec((ne, 1), lambda i: (0, 0)),
        ],
        out_specs=[
            pl.BlockSpec((ne, tile), lambda i: (0, i)),
            pl.BlockSpec((ne, tile), lambda i: (0, i)),
        ],
    )
    pipeline(x_hbm, w_hbm, b_hbm, logits_hbm, probs_hbm)


def kernel(input, W, b):
    import functools
    n, d = input.shape
    num_experts = W.shape[0]
    tile = 2048
    b2 = b.reshape(num_experts, 1)
    outer = functools.partial(_outer, tile=tile, n=n, d=d, ne=num_experts)
    logits_t, probs_t = pl.pallas_call(
        outer,
        in_specs=[
            pl.BlockSpec(memory_space=pl.ANY),
            pl.BlockSpec(memory_space=pl.ANY),
            pl.BlockSpec(memory_space=pl.ANY),
        ],
        out_specs=[
            pl.BlockSpec(memory_space=pl.ANY),
            pl.BlockSpec(memory_space=pl.ANY),
        ],
        out_shape=[
            jax.ShapeDtypeStruct((num_experts, n), jnp.float32),
            jax.ShapeDtypeStruct((num_experts, n), jnp.float32),
        ],
    )(input, W, b2)
    return (logits_t.T, probs_t.T)
